# baseline (speedup 1.0000x reference)
import jax
import jax.numpy as jnp
from jax import lax
from jax.experimental import pallas as pl
from jax.experimental.pallas import tpu as pltpu

_B, _K = 128, 8


def _tc_body(acc_ref, sub_ref, draft_ref, bonus_ref, out_ref, na_ref, ng_ref,
             ne_ref):
    a = acc_ref[...].astype(jnp.int32)
    sub = sub_ref[...]
    draft = draft_ref[...]
    row = lax.broadcasted_iota(jnp.int32, (_K, _B), 0)

    def shiftk(x, d):
        return jnp.where(row >= d, pltpu.roll(x, d, 0), 0)

    bad = 1 - a
    s = bad | shiftk(bad, 1)
    s = s | shiftk(s, 2)
    incl = s | shiftk(s, 4)
    acc_mask = 1 - incl
    after = bad * (1 - shiftk(incl, 1))

    out8 = acc_mask * draft + after * sub + (acc_mask + after - 1)
    pref = acc_mask[_K - 1:_K, :]
    bonus = bonus_ref[...]
    ob = pref * bonus + (pref - 1)
    out_ref[...] = jnp.concatenate([out8, ob], axis=0)

    na_ref[0, 0] = jnp.sum(a)
    ng = jnp.sum(acc_mask)
    ng_ref[0, 0] = ng
    ne_ref[0, 0] = ng + _B


def kernel(accepted, substitute_token_ids, draft_token_ids, bonus_token_ids,
           total_num_seqs):
    del total_num_seqs
    out_t, na, ng, ne = pl.pallas_call(
        _tc_body,
        out_shape=[
            jax.ShapeDtypeStruct((_K + 1, _B), jnp.int32),
            jax.ShapeDtypeStruct((1, 1), jnp.int32),
            jax.ShapeDtypeStruct((1, 1), jnp.int32),
            jax.ShapeDtypeStruct((1, 1), jnp.int32),
        ],
        out_specs=[
            pl.BlockSpec(memory_space=pltpu.VMEM),
            pl.BlockSpec(memory_space=pltpu.SMEM),
            pl.BlockSpec(memory_space=pltpu.SMEM),
            pl.BlockSpec(memory_space=pltpu.SMEM),
        ],
        name="spec_decode_sampler_patch_tc3",
    )(accepted.view(jnp.int8).T,
      substitute_token_ids.T, draft_token_ids.T, bonus_token_ids.T)
    return out_t.T, na[0, 0], ng[0, 0], ne[0, 0]


# speedup vs baseline: 1.2054x; 1.2054x over previous
import jax
import jax.numpy as jnp
from jax import lax
from jax.experimental import pallas as pl
from jax.experimental.pallas import tpu as pltpu

_B, _K = 128, 8


def _tc_body(acc_ref, sub_ref, draft_ref, bonus_ref, out_ref, na_ref, ng_ref,
             ne_ref):
    a = acc_ref[...].astype(jnp.int32)
    sub = sub_ref[...]
    draft = draft_ref[...]
    row = lax.broadcasted_iota(jnp.int32, (_K, _B), 0)

    def shiftk(x, d):
        return jnp.where(row >= d, pltpu.roll(x, d, 0), 0)

    bad = 1 - a
    s = bad | shiftk(bad, 1)
    s = s | shiftk(s, 2)
    incl = s | shiftk(s, 4)
    acc_mask = 1 - incl
    after = bad * (1 - shiftk(incl, 1))

    out8 = acc_mask * draft + after * sub + (acc_mask + after - 1)
    pref = acc_mask[_K - 1:_K, :]
    bonus = bonus_ref[...]
    ob = pref * bonus + (pref - 1)
    out_ref[...] = jnp.concatenate([out8, ob], axis=0)

    na_ref[0, 0] = jnp.sum(a)
    ng = jnp.sum(acc_mask)
    ng_ref[0, 0] = ng
    ne_ref[0, 0] = ng + _B


def kernel(accepted, substitute_token_ids, draft_token_ids, bonus_token_ids,
           total_num_seqs):
    del total_num_seqs
    out_t, na, ng, ne = pl.pallas_call(
        _tc_body,
        out_shape=[
            jax.ShapeDtypeStruct((_K + 1, _B), jnp.int32),
            jax.ShapeDtypeStruct((1, 1), jnp.int32),
            jax.ShapeDtypeStruct((1, 1), jnp.int32),
            jax.ShapeDtypeStruct((1, 1), jnp.int32),
        ],
        out_specs=[
            pl.BlockSpec(memory_space=pltpu.VMEM),
            pl.BlockSpec(memory_space=pltpu.SMEM),
            pl.BlockSpec(memory_space=pltpu.SMEM),
            pl.BlockSpec(memory_space=pltpu.SMEM),
        ],
        compiler_params=pltpu.CompilerParams(
            allow_input_fusion=[True, False, False, False]),
        name="spec_decode_sampler_patch_tc3",
    )(accepted.view(jnp.int8).T,
      substitute_token_ids.T, draft_token_ids.T, bonus_token_ids.T)
    return out_t.T, na[0, 0], ng[0, 0], ne[0, 0]


# speedup vs baseline: 1.5235x; 1.2639x over previous
import jax
import jax.numpy as jnp
from jax import lax
from jax.experimental import pallas as pl
from jax.experimental.pallas import tpu as pltpu

_B, _K = 128, 8


def _tc_body(acc_ref, sub_ref, draft_ref, bonus_ref, out_ref, na_ref, ng_ref,
             ne_ref):
    a = acc_ref[...].astype(jnp.int32)
    sub = sub_ref[...]
    draft = draft_ref[...]
    row = lax.broadcasted_iota(jnp.int32, (_K, _B), 0)

    def shiftk(x, d):
        return jnp.where(row >= d, pltpu.roll(x, d, 0), 0)

    bad = 1 - a
    s = bad | shiftk(bad, 1)
    s = s | shiftk(s, 2)
    incl = s | shiftk(s, 4)
    acc_mask = 1 - incl
    after = bad * (1 - shiftk(incl, 1))

    out8 = acc_mask * draft + after * sub + (acc_mask + after - 1)
    pref = acc_mask[_K - 1:_K, :]
    bonus = bonus_ref[...]
    ob = pref * bonus + (pref - 1)
    out_ref[...] = jnp.concatenate([out8, ob], axis=0)

    na_ref[0, 0] = jnp.sum(a)
    ng = jnp.sum(acc_mask)
    ng_ref[0, 0] = ng
    ne_ref[0, 0] = ng + _B


def kernel(accepted, substitute_token_ids, draft_token_ids, bonus_token_ids,
           total_num_seqs):
    del total_num_seqs
    out_t, na, ng, ne = pl.pallas_call(
        _tc_body,
        out_shape=[
            jax.ShapeDtypeStruct((_K + 1, _B), jnp.int32),
            jax.ShapeDtypeStruct((1, 1), jnp.int32),
            jax.ShapeDtypeStruct((1, 1), jnp.int32),
            jax.ShapeDtypeStruct((1, 1), jnp.int32),
        ],
        out_specs=[
            pl.BlockSpec(memory_space=pltpu.VMEM),
            pl.BlockSpec(memory_space=pltpu.SMEM),
            pl.BlockSpec(memory_space=pltpu.SMEM),
            pl.BlockSpec(memory_space=pltpu.SMEM),
        ],
        compiler_params=pltpu.CompilerParams(
            allow_input_fusion=[True, True, True, True]),
        name="spec_decode_sampler_patch_tc3",
    )(accepted.view(jnp.int8).T,
      substitute_token_ids.T, draft_token_ids.T, bonus_token_ids.T)
    return out_t.T, na[0, 0], ng[0, 0], ne[0, 0]
